# Initial kernel scaffold; baseline (speedup 1.0000x reference)
import jax
import jax.numpy as jnp
from jax.experimental import pallas as pl


def kernel(hidden_states, gate_w, w1, w2, w3, ws1, ws2, ws3, shared_gate_w):
    raise NotImplementedError("write your pallas kernel here")


# speedup vs baseline: 1.6530x; 1.6530x over previous
import functools

import jax
import jax.numpy as jnp
from jax.experimental import pallas as pl
from jax.experimental.pallas import tpu as pltpu

NUM_EXPERTS = 16
TOP_K = 2
HIDDEN = 1024
MOE_FF = 512
T_TILE = 512


def _silu(x):
    return x * jax.nn.sigmoid(x)


def _router_body(x_ref, gate_ref, comb_ref):
    x = x_ref[...]
    logits = jax.lax.dot_general(
        x, gate_ref[...], (((1,), (1,)), ((), ())),
        preferred_element_type=jnp.float32)
    m = jnp.max(logits, axis=1, keepdims=True)
    ex = jnp.exp(logits - m)
    sm = ex / jnp.sum(ex, axis=1, keepdims=True)
    lane = jax.lax.broadcasted_iota(jnp.int32, sm.shape, 1)
    m1 = jnp.max(sm, axis=1, keepdims=True)
    i1 = jnp.min(jnp.where(sm == m1, lane, NUM_EXPERTS), axis=1, keepdims=True)
    oh1 = (lane == i1)
    sm2 = jnp.where(oh1, -jnp.inf, sm)
    m2 = jnp.max(sm2, axis=1, keepdims=True)
    i2 = jnp.min(jnp.where(sm2 == m2, lane, NUM_EXPERTS), axis=1, keepdims=True)
    oh2 = (lane == i2)
    denom = m1 + m2
    comb_ref[...] = jnp.where(oh1, m1 / denom, 0.0) + jnp.where(oh2, m2 / denom, 0.0)


def _moe_body(x_ref, w1_ref, w2_ref, w3_ref, comb_ref, out_ref, acc_ref):
    e = pl.program_id(1)
    xb = x_ref[...].astype(jnp.bfloat16)
    w1 = w1_ref[0].astype(jnp.bfloat16)
    w3 = w3_ref[0].astype(jnp.bfloat16)
    w2 = w2_ref[0].astype(jnp.bfloat16)
    h = jax.lax.dot_general(xb, w1, (((1,), (1,)), ((), ())),
                            preferred_element_type=jnp.float32)
    u = jax.lax.dot_general(xb, w3, (((1,), (1,)), ((), ())),
                            preferred_element_type=jnp.float32)
    act = (_silu(h) * u).astype(jnp.bfloat16)
    y = jax.lax.dot_general(act, w2, (((1,), (1,)), ((), ())),
                            preferred_element_type=jnp.float32)
    lane = jax.lax.broadcasted_iota(jnp.int32, comb_ref.shape, 1)
    we = jnp.sum(jnp.where(lane == e, comb_ref[...], 0.0), axis=1, keepdims=True)
    contrib = we * y

    @pl.when(e == 0)
    def _():
        acc_ref[...] = contrib

    @pl.when(e > 0)
    def _():
        acc_ref[...] += contrib

    @pl.when(e == NUM_EXPERTS - 1)
    def _():
        out_ref[...] = acc_ref[...]


def _shared_body(x_ref, ws1_ref, ws2_ref, ws3_ref, sg_ref, moe_ref, out_ref):
    x = x_ref[...]
    xb = x.astype(jnp.bfloat16)
    h = jax.lax.dot_general(xb, ws1_ref[...].astype(jnp.bfloat16),
                            (((1,), (1,)), ((), ())),
                            preferred_element_type=jnp.float32)
    u = jax.lax.dot_general(xb, ws3_ref[...].astype(jnp.bfloat16),
                            (((1,), (1,)), ((), ())),
                            preferred_element_type=jnp.float32)
    act = (_silu(h) * u).astype(jnp.bfloat16)
    shared = jax.lax.dot_general(act, ws2_ref[...].astype(jnp.bfloat16),
                                 (((1,), (1,)), ((), ())),
                                 preferred_element_type=jnp.float32)
    g = jax.nn.sigmoid(jax.lax.dot_general(
        x, sg_ref[...], (((1,), (1,)), ((), ())),
        preferred_element_type=jnp.float32))
    out_ref[...] = moe_ref[...] + g * shared


def kernel(hidden_states, gate_w, w1, w2, w3, ws1, ws2, ws3, shared_gate_w):
    B, S, D = hidden_states.shape
    x = hidden_states.reshape(-1, D)
    T = x.shape[0]

    combine = pl.pallas_call(
        _router_body,
        out_shape=jax.ShapeDtypeStruct((T, NUM_EXPERTS), jnp.float32),
    )(x, gate_w)

    nt = T // T_TILE
    moe_out = pl.pallas_call(
        _moe_body,
        grid=(nt, NUM_EXPERTS),
        in_specs=[
            pl.BlockSpec((T_TILE, D), lambda t, e: (t, 0)),
            pl.BlockSpec((1, MOE_FF, D), lambda t, e: (e, 0, 0)),
            pl.BlockSpec((1, D, MOE_FF), lambda t, e: (e, 0, 0)),
            pl.BlockSpec((1, MOE_FF, D), lambda t, e: (e, 0, 0)),
            pl.BlockSpec((T_TILE, NUM_EXPERTS), lambda t, e: (t, 0)),
        ],
        out_specs=pl.BlockSpec((T_TILE, D), lambda t, e: (t, 0)),
        out_shape=jax.ShapeDtypeStruct((T, D), jnp.float32),
        scratch_shapes=[pltpu.VMEM((T_TILE, D), jnp.float32)],
        compiler_params=pltpu.CompilerParams(
            dimension_semantics=("arbitrary", "arbitrary")),
    )(x, w1, w2, w3, combine)

    out = pl.pallas_call(
        _shared_body,
        grid=(nt,),
        in_specs=[
            pl.BlockSpec((T_TILE, D), lambda t: (t, 0)),
            pl.BlockSpec((MOE_FF, D), lambda t: (0, 0)),
            pl.BlockSpec((D, MOE_FF), lambda t: (0, 0)),
            pl.BlockSpec((MOE_FF, D), lambda t: (0, 0)),
            pl.BlockSpec((1, D), lambda t: (0, 0)),
            pl.BlockSpec((T_TILE, D), lambda t: (t, 0)),
        ],
        out_specs=pl.BlockSpec((T_TILE, D), lambda t: (t, 0)),
        out_shape=jax.ShapeDtypeStruct((T, D), jnp.float32),
    )(x, ws1, ws2, ws3, shared_gate_w, moe_out)

    return out.reshape(B, S, D)
